# baseline (speedup 1.0000x reference)
import jax
import jax.numpy as jnp
from jax import lax
from jax.experimental import pallas as pl
from jax.experimental.pallas import tpu as pltpu
from jax.experimental.pallas import tpu_sc as plsc

GRID_H, GRID_W, EMBED_DIM = 32, 32, 768
D = EMBED_DIM // 3
N = GRID_H * GRID_W

NC, NS = 2, 16
NW = NC * NS
L = 16


def _sc_body(row_hbm, col_hbm, time_hbm, cls_hbm, out_hbm,
             row_v, col_v, time_v, cls_v, idx_v, sem):
    wid = lax.axis_index("s") * NC + lax.axis_index("c")
    base = 1 + GRID_W * wid

    widv = jnp.full((L,), wid, jnp.int32)
    idx_v[pl.ds(0, L)] = widv
    idx_v[pl.ds(L, L)] = widv

    pltpu.async_copy(row_hbm.at[idx_v], row_v, sem).wait()
    pltpu.sync_copy(col_hbm, col_v)
    pltpu.sync_copy(time_hbm.at[pl.ds(GRID_W * wid, GRID_W)], time_v)

    pltpu.sync_copy(row_v, out_hbm.at[pl.ds(base, GRID_W), pl.ds(0, D)])
    pltpu.sync_copy(col_v, out_hbm.at[pl.ds(base, GRID_W), pl.ds(D, D)])
    pltpu.sync_copy(time_v, out_hbm.at[pl.ds(base, GRID_W), pl.ds(2 * D, D)])

    @pl.when(wid == 0)
    def _():
        pltpu.sync_copy(cls_hbm, cls_v)
        pltpu.sync_copy(cls_v, out_hbm.at[pl.ds(0, 1)])


def kernel(x, row_embed, col_embed, time_embed, cls_token_pos):
    mesh = plsc.VectorSubcoreMesh(core_axis_name="c", subcore_axis_name="s")
    cls2d = cls_token_pos.reshape(1, EMBED_DIM)
    run = pl.kernel(
        _sc_body,
        mesh=mesh,
        out_type=jax.ShapeDtypeStruct((N + 1, EMBED_DIM), jnp.float32),
        scratch_types=[
            pltpu.VMEM((GRID_W, D), jnp.float32),
            pltpu.VMEM((GRID_W, D), jnp.float32),
            pltpu.VMEM((GRID_W, D), jnp.float32),
            pltpu.VMEM((1, EMBED_DIM), jnp.float32),
            pltpu.VMEM((GRID_W,), jnp.int32),
            pltpu.SemaphoreType.DMA,
        ],
        compiler_params=pltpu.CompilerParams(use_tc_tiling_on_sc=False),
    )
    out = run(row_embed, col_embed, time_embed, cls2d)
    return out[None]


# speedup vs baseline: 1.0588x; 1.0588x over previous
import jax
import jax.numpy as jnp
from jax import lax
from jax.experimental import pallas as pl
from jax.experimental.pallas import tpu as pltpu
from jax.experimental.pallas import tpu_sc as plsc

GRID_H, GRID_W, EMBED_DIM = 32, 32, 768
D = EMBED_DIM // 3
N = GRID_H * GRID_W

NC, NS = 2, 16
NW = NC * NS
L = 16


def _sc_body(row_hbm, col_hbm, time_hbm, cls_hbm, out_hbm,
             row_v, col_v, time_v, cls_v, idx_v, lsem, ssem):
    wid = lax.axis_index("s") * NC + lax.axis_index("c")
    base = 1 + GRID_W * wid

    widv = jnp.full((L,), wid, jnp.int32)
    idx_v[pl.ds(0, L)] = widv
    idx_v[pl.ds(L, L)] = widv

    c_row = pltpu.async_copy(row_hbm.at[idx_v], row_v, lsem)
    c_col = pltpu.async_copy(col_hbm, col_v, lsem)
    c_time = pltpu.async_copy(time_hbm.at[pl.ds(GRID_W * wid, GRID_W)],
                              time_v, lsem)
    c_row.wait()
    c_col.wait()
    c_time.wait()

    s_row = pltpu.async_copy(
        row_v, out_hbm.at[pl.ds(base, GRID_W), pl.ds(0, D)], ssem)
    s_col = pltpu.async_copy(
        col_v, out_hbm.at[pl.ds(base, GRID_W), pl.ds(D, D)], ssem)
    s_time = pltpu.async_copy(
        time_v, out_hbm.at[pl.ds(base, GRID_W), pl.ds(2 * D, D)], ssem)

    @pl.when(wid == 0)
    def _():
        pltpu.sync_copy(cls_hbm, cls_v)
        pltpu.sync_copy(cls_v, out_hbm.at[pl.ds(0, 1)])

    s_row.wait()
    s_col.wait()
    s_time.wait()


def kernel(x, row_embed, col_embed, time_embed, cls_token_pos):
    mesh = plsc.VectorSubcoreMesh(core_axis_name="c", subcore_axis_name="s")
    cls2d = cls_token_pos.reshape(1, EMBED_DIM)
    run = pl.kernel(
        _sc_body,
        mesh=mesh,
        out_type=jax.ShapeDtypeStruct((N + 1, EMBED_DIM), jnp.float32),
        scratch_types=[
            pltpu.VMEM((GRID_W, D), jnp.float32),
            pltpu.VMEM((GRID_W, D), jnp.float32),
            pltpu.VMEM((GRID_W, D), jnp.float32),
            pltpu.VMEM((1, EMBED_DIM), jnp.float32),
            pltpu.VMEM((GRID_W,), jnp.int32),
            pltpu.SemaphoreType.DMA,
            pltpu.SemaphoreType.DMA,
        ],
        compiler_params=pltpu.CompilerParams(use_tc_tiling_on_sc=False),
    )
    out = run(row_embed, col_embed, time_embed, cls2d)
    return out[None]


# speedup vs baseline: 1.1487x; 1.0849x over previous
import jax
import jax.numpy as jnp
from jax import lax
from jax.experimental import pallas as pl
from jax.experimental.pallas import tpu as pltpu
from jax.experimental.pallas import tpu_sc as plsc

GRID_H, GRID_W, EMBED_DIM = 32, 32, 768
D = EMBED_DIM // 3
N = GRID_H * GRID_W

NC, NS = 2, 16
NW = NC * NS
L = 16


def _sc_body(row_hbm, col_hbm, time_hbm, cls_hbm, out_hbm,
             row_v, col_v, time_v, cls_v, ridx_v, cidx_v, tidx_v,
             lsem, ssem):
    wid = lax.axis_index("s") * NC + lax.axis_index("c")
    base = GRID_W * wid

    iota = lax.iota(jnp.int32, L)

    lo = jnp.where(iota == 0, jnp.maximum(wid - 1, 0), wid)
    ridx_v[pl.ds(0, L)] = lo
    ridx_v[pl.ds(L, L)] = jnp.full((L,), wid, jnp.int32)

    cidx_v[pl.ds(0, L)] = jnp.where(iota == 0, GRID_W - 1, iota - 1)
    cidx_v[pl.ds(L, L)] = iota + (L - 1)

    tlo = jnp.maximum(iota + (GRID_W * wid - 1), 0)
    tidx_v[pl.ds(0, L)] = tlo
    tidx_v[pl.ds(L, L)] = iota + (GRID_W * wid - 1 + L)

    c_row = pltpu.async_copy(row_hbm.at[ridx_v], row_v, lsem)
    c_col = pltpu.async_copy(col_hbm.at[cidx_v], col_v, lsem)
    c_time = pltpu.async_copy(time_hbm.at[tidx_v], time_v, lsem)
    c_row.wait()
    c_col.wait()
    c_time.wait()

    s_row = pltpu.async_copy(
        row_v, out_hbm.at[pl.ds(base, GRID_W), pl.ds(0, D)], ssem)
    s_col = pltpu.async_copy(
        col_v, out_hbm.at[pl.ds(base, GRID_W), pl.ds(D, D)], ssem)
    s_time = pltpu.async_copy(
        time_v, out_hbm.at[pl.ds(base, GRID_W), pl.ds(2 * D, D)], ssem)
    s_row.wait()
    s_col.wait()
    s_time.wait()

    @pl.when(wid == 0)
    def _():
        pltpu.sync_copy(cls_hbm, cls_v)
        pltpu.sync_copy(cls_v, out_hbm.at[pl.ds(0, 1)])


def kernel(x, row_embed, col_embed, time_embed, cls_token_pos):
    mesh = plsc.VectorSubcoreMesh(core_axis_name="c", subcore_axis_name="s")
    cls2d = cls_token_pos.reshape(1, EMBED_DIM)
    run = pl.kernel(
        _sc_body,
        mesh=mesh,
        out_type=jax.ShapeDtypeStruct((N + 1, EMBED_DIM), jnp.float32),
        scratch_types=[
            pltpu.VMEM((GRID_W, D), jnp.float32),
            pltpu.VMEM((GRID_W, D), jnp.float32),
            pltpu.VMEM((GRID_W, D), jnp.float32),
            pltpu.VMEM((1, EMBED_DIM), jnp.float32),
            pltpu.VMEM((GRID_W,), jnp.int32),
            pltpu.VMEM((GRID_W,), jnp.int32),
            pltpu.VMEM((GRID_W,), jnp.int32),
            pltpu.SemaphoreType.DMA,
            pltpu.SemaphoreType.DMA,
        ],
    )
    out = run(row_embed, col_embed, time_embed, cls2d)
    return out[None]


# speedup vs baseline: 3.2222x; 2.8051x over previous
import jax
import jax.numpy as jnp
from jax.experimental import pallas as pl

GRID_H, GRID_W, EMBED_DIM = 32, 32, 768
D = EMBED_DIM // 3
N = GRID_H * GRID_W


def _pos_emb_kernel(row_ref, col_ref, time_ref, cls_ref, out_ref):
    row_grid = jnp.broadcast_to(row_ref[...][:, None, :], (GRID_H, GRID_W, D))
    col_grid = jnp.broadcast_to(col_ref[...][None, :, :], (GRID_H, GRID_W, D))
    out_ref[pl.ds(1, N), 0:D] = row_grid.reshape(N, D)
    out_ref[pl.ds(1, N), D:2 * D] = col_grid.reshape(N, D)
    out_ref[pl.ds(1, N), 2 * D:3 * D] = time_ref[...]
    out_ref[0:1, :] = cls_ref[0]


def kernel(x, row_embed, col_embed, time_embed, cls_token_pos):
    out = pl.pallas_call(
        _pos_emb_kernel,
        out_shape=jax.ShapeDtypeStruct((N + 1, EMBED_DIM), jnp.float32),
    )(row_embed, col_embed, time_embed, cls_token_pos)
    return out[None]


# speedup vs baseline: 3.2441x; 1.0068x over previous
import jax
import jax.numpy as jnp
from jax.experimental import pallas as pl

GRID_H, GRID_W, EMBED_DIM = 32, 32, 768
D = EMBED_DIM // 3
N = GRID_H * GRID_W


def _pos_emb_kernel(row_ref, col_ref, time_ref, cls_ref, out_ref):
    row_grid = jnp.broadcast_to(row_ref[...][:, None, :], (GRID_H, GRID_W, D))
    col_grid = jnp.broadcast_to(col_ref[...][None, :, :], (GRID_H, GRID_W, D))
    out_ref[0, pl.ds(1, N), 0:D] = row_grid.reshape(N, D)
    out_ref[0, pl.ds(1, N), D:2 * D] = col_grid.reshape(N, D)
    out_ref[0, pl.ds(1, N), 2 * D:3 * D] = time_ref[...]
    out_ref[0, 0:1, :] = cls_ref[0]


def kernel(x, row_embed, col_embed, time_embed, cls_token_pos):
    return pl.pallas_call(
        _pos_emb_kernel,
        out_shape=jax.ShapeDtypeStruct((1, N + 1, EMBED_DIM), jnp.float32),
    )(row_embed, col_embed, time_embed, cls_token_pos)


# speedup vs baseline: 10.2799x; 3.1688x over previous
import jax
import jax.numpy as jnp
from jax.experimental import pallas as pl

GRID_H, GRID_W, EMBED_DIM = 32, 32, 768
D = EMBED_DIM // 3
N = GRID_H * GRID_W


def _pos_emb_kernel(row_ref, col_ref, time_ref, cls_ref, out_ref):
    row_grid = jnp.broadcast_to(row_ref[...][:, None, :], (GRID_H, GRID_W, D))
    col_grid = jnp.broadcast_to(col_ref[...][None, :, :], (GRID_H, GRID_W, D))
    out_ref[pl.ds(1, N), 0, 0:D] = row_grid.reshape(N, D)
    out_ref[pl.ds(1, N), 0, D:2 * D] = col_grid.reshape(N, D)
    out_ref[pl.ds(1, N), 0, 2 * D:3 * D] = time_ref[...]
    out_ref[0:1, 0, :] = cls_ref[0]


def kernel(x, row_embed, col_embed, time_embed, cls_token_pos):
    out = pl.pallas_call(
        _pos_emb_kernel,
        out_shape=jax.ShapeDtypeStruct((N + 1, 1, EMBED_DIM), jnp.float32),
    )(row_embed, col_embed, time_embed, cls_token_pos)
    return out.reshape(1, N + 1, EMBED_DIM)
